# baseline (speedup 1.0000x reference)
import functools

import jax
import jax.numpy as jnp
from jax import lax
from jax.experimental import pallas as pl
from jax.experimental.pallas import tpu as pltpu
from jax.experimental.pallas import tpu_sc as plsc

NUM_CORES = 2
NUM_SUBCORES = 16
NUM_WORKERS = NUM_CORES * NUM_SUBCORES
NBUF = 8
GAHEAD = 4


@functools.partial(jax.jit, static_argnames=())
def _sc_embedding_gather(idx2, table):
    B, L = idx2.shape
    D = table.shape[1]
    rpw = B // NUM_WORKERS
    mesh = plsc.VectorSubcoreMesh(core_axis_name="c", subcore_axis_name="s")

    @functools.partial(
        pl.kernel,
        mesh=mesh,
        out_type=jax.ShapeDtypeStruct((B, L, D), jnp.float32),
        scratch_types=[
            pltpu.VMEM((rpw, L), jnp.int32),
            pltpu.VMEM((NBUF, L, D), jnp.float32),
            pltpu.SemaphoreType.DMA((NBUF,)),
            pltpu.SemaphoreType.DMA((NBUF,)),
            pltpu.SemaphoreType.DMA,
        ],
        compiler_params=pltpu.CompilerParams(use_tc_tiling_on_sc=False),
    )
    def k(idx_hbm, table_hbm, out_hbm, idx_v, bufs, gsem, osem, isem):
        wid = lax.axis_index("s") * NUM_CORES + lax.axis_index("c")
        base = wid * rpw
        pltpu.async_copy(
            idx_hbm.at[pl.ds(base, rpw)], idx_v, isem).wait()

        def gather(j, b):
            return pltpu.make_async_copy(table_hbm.at[idx_v.at[j]],
                                         bufs.at[b], gsem.at[b])

        def out_copy(j, b):
            return pltpu.make_async_copy(bufs.at[b], out_hbm.at[base + j],
                                         osem.at[b])

        for b in range(GAHEAD):
            gather(b, b).start()

        def body(j, carry):
            b = lax.rem(j, NBUF)
            gather(j, b).wait()
            out_copy(j, b).start()
            nxt = j + GAHEAD

            @pl.when(nxt < rpw)
            def _():
                nb = lax.rem(nxt, NBUF)

                @pl.when(nxt >= NBUF)
                def _():
                    out_copy(nxt - NBUF, nb).wait()

                gather(nxt, nb).start()

            return carry

        lax.fori_loop(0, rpw, body, 0)

        for i in range(NBUF):
            j = rpw - NBUF + i
            out_copy(j, j % NBUF).wait()

    return k(idx2, table)


def kernel(token_ids, embedding):
    B, L = token_ids.shape
    idx2 = token_ids.astype(jnp.int32)
    b_total = max(-(-B // NUM_WORKERS), NBUF) * NUM_WORKERS
    if b_total != B:
        pad = jnp.zeros((b_total - B, L), jnp.int32)
        idx2 = jnp.concatenate([idx2, pad])
    out = _sc_embedding_gather(idx2, embedding)
    return out[:B]


# speedup vs baseline: 1.0087x; 1.0087x over previous
import functools

import jax
import jax.numpy as jnp
from jax import lax
from jax.experimental import pallas as pl
from jax.experimental.pallas import tpu as pltpu
from jax.experimental.pallas import tpu_sc as plsc

NUM_CORES = 2
NUM_SUBCORES = 16
NUM_WORKERS = NUM_CORES * NUM_SUBCORES
NBUF = 8
GAHEAD = 6


@functools.partial(jax.jit, static_argnames=())
def _sc_embedding_gather(idx2, table):
    B, L = idx2.shape
    D = table.shape[1]
    rpw = B // NUM_WORKERS
    mesh = plsc.VectorSubcoreMesh(core_axis_name="c", subcore_axis_name="s")

    @functools.partial(
        pl.kernel,
        mesh=mesh,
        out_type=jax.ShapeDtypeStruct((B, L, D), jnp.float32),
        scratch_types=[
            pltpu.VMEM((rpw, L), jnp.int32),
            pltpu.VMEM((NBUF, L, D), jnp.float32),
            pltpu.SemaphoreType.DMA((NBUF,)),
            pltpu.SemaphoreType.DMA((NBUF,)),
            pltpu.SemaphoreType.DMA,
        ],
        compiler_params=pltpu.CompilerParams(use_tc_tiling_on_sc=False),
    )
    def k(idx_hbm, table_hbm, out_hbm, idx_v, bufs, gsem, osem, isem):
        wid = lax.axis_index("s") * NUM_CORES + lax.axis_index("c")
        base = wid * rpw
        pltpu.async_copy(
            idx_hbm.at[pl.ds(base, rpw)], idx_v, isem).wait()

        def gather(j, b):
            return pltpu.make_async_copy(table_hbm.at[idx_v.at[j]],
                                         bufs.at[b], gsem.at[b])

        def out_copy(j, b):
            return pltpu.make_async_copy(bufs.at[b], out_hbm.at[base + j],
                                         osem.at[b])

        for b in range(GAHEAD):
            gather(b, b).start()

        def body(j, carry):
            b = lax.rem(j, NBUF)
            gather(j, b).wait()
            out_copy(j, b).start()
            nxt = j + GAHEAD

            @pl.when(nxt < rpw)
            def _():
                nb = lax.rem(nxt, NBUF)

                @pl.when(nxt >= NBUF)
                def _():
                    out_copy(nxt - NBUF, nb).wait()

                gather(nxt, nb).start()

            return carry

        lax.fori_loop(0, rpw, body, 0)

        for i in range(NBUF):
            j = rpw - NBUF + i
            out_copy(j, j % NBUF).wait()

    return k(idx2, table)


def kernel(token_ids, embedding):
    B, L = token_ids.shape
    idx2 = token_ids.astype(jnp.int32)
    b_total = max(-(-B // NUM_WORKERS), NBUF) * NUM_WORKERS
    if b_total != B:
        pad = jnp.zeros((b_total - B, L), jnp.int32)
        idx2 = jnp.concatenate([idx2, pad])
    out = _sc_embedding_gather(idx2, embedding)
    return out[:B]
